# baseline (speedup 1.0000x reference)
import functools

import jax
import jax.numpy as jnp
from jax import lax
from jax.experimental import pallas as pl
from jax.experimental.pallas import tpu as pltpu
from jax.experimental.pallas import tpu_sc as plsc

_VOCAB = 1000
_CODE = 64
_LATENT = 512
_NC = 2
_NS = 16
_NW = _NC * _NS
_C = 80
_NCHUNK = 20
_NBUF = 2
_B_PER_W = _C * _NCHUNK


def _proj_body(emb_ref, vv_ref, g_ref, b_ref, out_ref):
    vv = vv_ref[...]
    norm = jnp.sqrt(jnp.sum(vv * vv, axis=1, keepdims=True) + 1e-12)
    w2 = vv * (g_ref[...] / norm)
    out_ref[...] = lax.dot_general(
        emb_ref[...], w2, (((1,), (1,)), ((), ())),
        preferred_element_type=jnp.float32,
        precision=lax.Precision.HIGHEST,
    ) + b_ref[...]


def _project_table(emb_table, vv, g2, b2):
    return pl.pallas_call(
        _proj_body,
        out_shape=jax.ShapeDtypeStruct((_VOCAB, _LATENT), jnp.float32),
    )(emb_table, vv, g2, b2)


def _gather_body(proj_hbm, xf_hbm, out_hbm, idx_v, *scr):
    rows = scr[:_NBUF]
    gsem = scr[_NBUF:]
    wid = lax.axis_index("s") * _NC + lax.axis_index("c")
    pltpu.sync_copy(xf_hbm.at[wid], idx_v)
    base = wid * _B_PER_W

    for b in range(_NBUF):
        pltpu.make_async_copy(proj_hbm.at[idx_v.at[b]], rows[b], gsem[b]).start()

    def outer(j, carry):
        for b in range(_NBUF):
            k = j * _NBUF + b
            pltpu.make_async_copy(
                proj_hbm.at[idx_v.at[k]], rows[b], gsem[b]).wait()
            off = pl.multiple_of(base + k * _C, _C)
            pltpu.sync_copy(rows[b], out_hbm.at[pl.ds(off, _C)])
            kn = k + _NBUF

            @pl.when(kn < _NCHUNK)
            def _():
                pltpu.make_async_copy(
                    proj_hbm.at[idx_v.at[kn]], rows[b], gsem[b]).start()
        return carry

    lax.fori_loop(0, _NCHUNK // _NBUF, outer, 0)


@functools.cache
def _gather_call():
    return pl.kernel(
        _gather_body,
        mesh=plsc.VectorSubcoreMesh(
            core_axis_name="c", subcore_axis_name="s",
            num_cores=_NC, num_subcores=_NS,
        ),
        out_type=jax.ShapeDtypeStruct((_NW * _B_PER_W, _LATENT), jnp.float32),
        scratch_types=(
            [pltpu.VMEM((_NCHUNK, _C), jnp.int32)]
            + [pltpu.VMEM((_C, _LATENT), jnp.float32) for _ in range(_NBUF)]
            + [pltpu.SemaphoreType.DMA for _ in range(_NBUF)]
        ),
    )


def kernel(x, emb_table, v, g, b):
    B, T = x.shape
    vv = v[:, :, 0]
    g2 = g[:, 0, :]
    b2 = b[None, :]
    proj = _project_table(emb_table, vv, g2, b2)
    xf = jnp.transpose(x.astype(jnp.int32)).reshape(_NW, _NCHUNK, _C)
    out = _gather_call()(proj, xf)
    return out.reshape(T, B, _LATENT).transpose(1, 0, 2)


# speedup vs baseline: 1.0092x; 1.0092x over previous
import functools

import jax
import jax.numpy as jnp
from jax import lax
from jax.experimental import pallas as pl
from jax.experimental.pallas import tpu as pltpu
from jax.experimental.pallas import tpu_sc as plsc

_VOCAB = 1000
_CODE = 64
_LATENT = 512
_NC = 2
_NS = 16
_NW = _NC * _NS
_C = 80
_NCHUNK = 20
_NBUF = 3
_B_PER_W = _C * _NCHUNK


def _proj_body(emb_ref, vv_ref, g_ref, b_ref, out_ref):
    vv = vv_ref[...]
    norm = jnp.sqrt(jnp.sum(vv * vv, axis=1, keepdims=True) + 1e-12)
    w2 = vv * (g_ref[...] / norm)
    out_ref[...] = lax.dot_general(
        emb_ref[...], w2, (((1,), (1,)), ((), ())),
        preferred_element_type=jnp.float32,
        precision=lax.Precision.HIGHEST,
    ) + b_ref[...]


def _project_table(emb_table, vv, g2, b2):
    return pl.pallas_call(
        _proj_body,
        out_shape=jax.ShapeDtypeStruct((_VOCAB, _LATENT), jnp.float32),
    )(emb_table, vv, g2, b2)


def _gather_body(proj_hbm, xf_hbm, out_hbm, idx_v, *scr):
    rows = scr[:_NBUF]
    gsem = scr[_NBUF:]
    wid = lax.axis_index("s") * _NC + lax.axis_index("c")
    pltpu.sync_copy(xf_hbm.at[wid], idx_v)
    base = wid * _B_PER_W

    for b in range(_NBUF):
        pltpu.make_async_copy(proj_hbm.at[idx_v.at[b]], rows[b], gsem[b]).start()

    def outer(j, carry):
        for b in range(_NBUF):
            k = j * _NBUF + b

            @pl.when(k < _NCHUNK)
            def _():
                pltpu.make_async_copy(
                    proj_hbm.at[idx_v.at[k]], rows[b], gsem[b]).wait()
                off = pl.multiple_of(base + k * _C, _C)
                pltpu.sync_copy(rows[b], out_hbm.at[pl.ds(off, _C)])
                kn = k + _NBUF

                @pl.when(kn < _NCHUNK)
                def _():
                    pltpu.make_async_copy(
                        proj_hbm.at[idx_v.at[kn]], rows[b], gsem[b]).start()
        return carry

    lax.fori_loop(0, -(-_NCHUNK // _NBUF), outer, 0)


@functools.cache
def _gather_call():
    return pl.kernel(
        _gather_body,
        mesh=plsc.VectorSubcoreMesh(
            core_axis_name="c", subcore_axis_name="s",
            num_cores=_NC, num_subcores=_NS,
        ),
        out_type=jax.ShapeDtypeStruct((_NW * _B_PER_W, _LATENT), jnp.float32),
        scratch_types=(
            [pltpu.VMEM((_NCHUNK, _C), jnp.int32)]
            + [pltpu.VMEM((_C, _LATENT), jnp.float32) for _ in range(_NBUF)]
            + [pltpu.SemaphoreType.DMA for _ in range(_NBUF)]
        ),
    )


def kernel(x, emb_table, v, g, b):
    B, T = x.shape
    vv = v[:, :, 0]
    g2 = g[:, 0, :]
    b2 = b[None, :]
    proj = _project_table(emb_table, vv, g2, b2)
    xf = jnp.transpose(x.astype(jnp.int32)).reshape(_NW, _NCHUNK, _C)
    out = _gather_call()(proj, xf)
    return out.reshape(T, B, _LATENT).transpose(1, 0, 2)
